# baseline (speedup 1.0000x reference)
import functools
import jax
import jax.numpy as jnp
from jax import lax
from jax.experimental import pallas as pl
from jax.experimental.pallas import tpu as pltpu
from jax.experimental.pallas import tpu_sc as plsc

D = 64
SCALE = 8.0

NUM_CORES = 2
NUM_SUBCORES = 16
NW = NUM_CORES * NUM_SUBCORES

BATCH = 4096
SEQ = 200
BW = BATCH // NW
VOCAB2 = 500000


def _encoder_fn():
    mesh = plsc.VectorSubcoreMesh(core_axis_name="c", subcore_axis_name="s")

    @functools.partial(
        pl.kernel,
        out_type=jax.ShapeDtypeStruct((SEQ, D, BATCH), jnp.float32),
        mesh=mesh,
        scratch_types=[
            pltpu.VMEM((BW,), jnp.int32),
            pltpu.VMEM((BW,), jnp.int32),
            pltpu.VMEM((BW,), jnp.int32),
            pltpu.VMEM((BW,), jnp.int32),
            pltpu.VMEM((BW, 2 * D), jnp.float32),
            pltpu.VMEM((BW, 2 * D), jnp.float32),
            pltpu.VMEM((D, BW), jnp.float32),
            pltpu.SemaphoreType.DMA,
            pltpu.SemaphoreType.DMA,
        ],
        compiler_params=pltpu.CompilerParams(
            use_tc_tiling_on_sc=True, needs_layout_passes=False),
    )
    def enc_kernel(xt_hbm, table2_hbm, out_hbm,
                   idx0, idx1, idx2a, idx2b, rows0, rows1, trans, sem0, sem1):
        wid = lax.axis_index("s") * NUM_CORES + lax.axis_index("c")
        b0 = wid * BW
        idxs = (idx0, idx1)
        idx2s = (idx2a, idx2b)
        rows = (rows0, rows1)
        sems = (sem0, sem1)

        def fire(s, slot):
            pltpu.sync_copy(xt_hbm.at[pl.ds(s * BATCH + b0, BW)], idxs[slot])
            for k in range(BW // 16):
                sl = pl.ds(k * 16, 16)
                idx2s[slot][sl] = lax.shift_right_logical(idxs[slot][sl], 1)
            pltpu.async_copy(table2_hbm.at[idx2s[slot]], rows[slot], sems[slot])

        def wait(slot):
            pltpu.make_async_copy(
                table2_hbm.at[idx2s[slot]], rows[slot], sems[slot]).wait()

        def transform_write(s, slot):
            for k in range(BW // 16):
                rids = lax.iota(jnp.int32, 16) + (k * 16)
                half = lax.shift_left(
                    jnp.bitwise_and(idxs[slot][pl.ds(k * 16, 16)], 1), 6)

                def col(d, _):
                    v = plsc.load_gather(rows[slot], [rids, half + d])
                    trans[d, pl.ds(k * 16, 16)] = v * SCALE
                    return 0

                lax.fori_loop(0, D, col, 0)
            pltpu.sync_copy(trans, out_hbm.at[s, :, pl.ds(b0, BW)])

        fire(0, 0)

        def pair_body(g, _):
            for b in range(2):
                s = 2 * g + b

                @pl.when(s + 1 < SEQ)
                def _():
                    fire(s + 1, 1 - b)

                wait(b)
                transform_write(s, b)
            return 0

        lax.fori_loop(0, SEQ // 2, pair_body, 0)

    return enc_kernel


_ENCODER = _encoder_fn()


def kernel(x, table):
    xt = jnp.transpose(x, (1, 0)).reshape(-1).astype(jnp.int32)
    table2 = table.reshape(VOCAB2, 2 * D)
    out_t = _ENCODER(xt, table2)
    return jnp.transpose(out_t, (0, 2, 1))
